# baseline (speedup 1.0000x reference)
import functools

import jax
import jax.numpy as jnp
from jax import lax
from jax.experimental import pallas as pl
from jax.experimental.pallas import tpu as pltpu
from jax.experimental.pallas import tpu_sc as plsc

_EPS = 1e-12
_HIDDEN = 128


_NC = 2
_NS = 16
_NW = _NC * _NS
_CHUNK = 64


_NBUF = 5


def _sc_gather(xidx, table):
    nw, n_chunks, chunk = xidx.shape
    tok = nw * n_chunks * chunk
    per_w = n_chunks * chunk
    mesh = plsc.VectorSubcoreMesh(core_axis_name="c", subcore_axis_name="s")

    @functools.partial(
        pl.kernel,
        mesh=mesh,
        out_type=jax.ShapeDtypeStruct((tok, _HIDDEN), jnp.float32),
        scratch_types=[
            pltpu.VMEM((n_chunks, chunk), jnp.int32),
            pltpu.VMEM((_NBUF, chunk, _HIDDEN), jnp.float32),
        ]
        + [pltpu.SemaphoreType.DMA] * (2 * _NBUF),
    )
    def gather_kernel(idx_hbm, table_hbm, out_hbm, idx_v, rows_v, *sems):
        gsem, wsem = sems[:_NBUF], sems[_NBUF:]
        wid = lax.axis_index("s") * _NC + lax.axis_index("c")
        base = pl.multiple_of(wid * per_w, chunk)

        pltpu.sync_copy(idx_hbm.at[wid], idx_v)

        def gather_chunk(c, b, start):
            cp = pltpu.make_async_copy(
                table_hbm.at[idx_v.at[c]], rows_v.at[b], gsem[b])
            cp.start() if start else cp.wait()

        def write_chunk(c, b, start):
            off = pl.multiple_of(base + c * chunk, chunk)
            cp = pltpu.make_async_copy(
                rows_v.at[b], out_hbm.at[pl.ds(off, chunk)], wsem[b])
            cp.start() if start else cp.wait()

        for b in range(_NBUF):
            gather_chunk(b, b, True)

        def round_(r, _):
            c0 = r * _NBUF
            for b in range(_NBUF):
                gather_chunk(c0 + b, b, False)
                write_chunk(c0 + b, b, True)
            for b in range(_NBUF):

                @pl.when(r < n_chunks // _NBUF - 1)
                def _():
                    write_chunk(c0 + b, b, False)
                    gather_chunk(c0 + _NBUF + b, b, True)

            return 0

        lax.fori_loop(0, n_chunks // _NBUF, round_, 0)
        for b in range(_NBUF):
            write_chunk(n_chunks - _NBUF + b, b, False)

    return gather_kernel(xidx, table)


_BK = 16
_SLICES = (256, 512, 256)


def _ln_body(g_ref, p_ref, gam_ref, bet_ref, o_ref):
    e = g_ref[...] + p_ref[...]
    m = jnp.mean(e, axis=-1, keepdims=True)
    d = e - m
    v = jnp.mean(d * d, axis=-1, keepdims=True)
    o_ref[...] = d * lax.rsqrt(v + _EPS) * gam_ref[...] + bet_ref[...]


def _tc_layernorm_slice(prev, gathered, pos, gamma, beta, block0, b):
    bs, s, h = gathered.shape

    data_specs = [
        pl.BlockSpec((_BK, s, h), lambda i: (i, 0, 0)),
        pl.BlockSpec((1, s, h), lambda i: (0, 0, 0)),
        pl.BlockSpec((1, 1, h), lambda i: (0, 0, 0)),
        pl.BlockSpec((1, 1, h), lambda i: (0, 0, 0)),
    ]
    common = dict(
        grid=(bs // _BK,),
        out_specs=pl.BlockSpec((_BK, s, h), lambda i: (block0 + i, 0, 0)),
        out_shape=jax.ShapeDtypeStruct((b, s, h), jnp.float32),
    )
    if prev is None:
        return pl.pallas_call(_ln_body, in_specs=data_specs, **common)(
            gathered, pos, gamma, beta)

    def body(_prev_ref, g_ref, p_ref, gam_ref, bet_ref, o_ref):
        _ln_body(g_ref, p_ref, gam_ref, bet_ref, o_ref)

    return pl.pallas_call(
        body,
        in_specs=[pl.BlockSpec((8, 8, h), lambda i: (0, 0, 0))] + data_specs,
        input_output_aliases={0: 0},
        **common,
    )(prev, gathered, pos, gamma, beta)


def kernel(x, word_table, pos_table, ln_gamma, ln_beta):
    b, s = x.shape
    pos = pos_table[:s][None]
    gamma = ln_gamma.reshape(1, 1, _HIDDEN)
    beta = ln_beta.reshape(1, 1, _HIDDEN)

    offs = [0]
    for bs in _SLICES:
        offs.append(offs[-1] + bs)
    gathered = [
        _sc_gather(
            x[offs[i]:offs[i + 1]].reshape(
                _NW, (bs * s) // (_NW * _CHUNK), _CHUNK),
            word_table,
        ).reshape(bs, s, _HIDDEN)
        for i, bs in enumerate(_SLICES)
    ]
    out = None
    for i, bs in enumerate(_SLICES):
        out = _tc_layernorm_slice(
            out, gathered[i], pos, gamma, beta, offs[i] // _BK, b)
    return out


# speedup vs baseline: 1.0622x; 1.0622x over previous
import functools

import jax
import jax.numpy as jnp
from jax import lax
from jax.experimental import pallas as pl
from jax.experimental.pallas import tpu as pltpu
from jax.experimental.pallas import tpu_sc as plsc

_EPS = 1e-12
_HIDDEN = 128


_NC = 2
_NS = 16
_NW = _NC * _NS
_CHUNK = 64


_NBUF = 5


def _sc_gather(xidx, table):
    nw, n_chunks, chunk = xidx.shape
    tok = nw * n_chunks * chunk
    per_w = n_chunks * chunk
    mesh = plsc.VectorSubcoreMesh(core_axis_name="c", subcore_axis_name="s")

    @functools.partial(
        pl.kernel,
        mesh=mesh,
        out_type=jax.ShapeDtypeStruct((tok, _HIDDEN), jnp.float32),
        scratch_types=[
            pltpu.VMEM((n_chunks, chunk), jnp.int32),
            pltpu.VMEM((_NBUF, chunk, _HIDDEN), jnp.float32),
        ]
        + [pltpu.SemaphoreType.DMA] * (2 * _NBUF),
    )
    def gather_kernel(idx_hbm, table_hbm, out_hbm, idx_v, rows_v, *sems):
        gsem, wsem = sems[:_NBUF], sems[_NBUF:]
        wid = lax.axis_index("s") * _NC + lax.axis_index("c")
        base = pl.multiple_of(wid * per_w, chunk)

        pltpu.sync_copy(idx_hbm.at[wid], idx_v)

        def gather_chunk(c, b, start):
            cp = pltpu.make_async_copy(
                table_hbm.at[idx_v.at[c]], rows_v.at[b], gsem[b])
            cp.start() if start else cp.wait()

        def write_chunk(c, b, start):
            off = pl.multiple_of(base + c * chunk, chunk)
            cp = pltpu.make_async_copy(
                rows_v.at[b], out_hbm.at[pl.ds(off, chunk)], wsem[b])
            cp.start() if start else cp.wait()

        for b in range(_NBUF):
            gather_chunk(b, b, True)

        def round_(r, _):
            c0 = r * _NBUF
            for b in range(_NBUF):
                gather_chunk(c0 + b, b, False)
                write_chunk(c0 + b, b, True)
            for b in range(_NBUF):

                @pl.when(r < n_chunks // _NBUF - 1)
                def _():
                    write_chunk(c0 + b, b, False)
                    gather_chunk(c0 + _NBUF + b, b, True)

            return 0

        lax.fori_loop(0, n_chunks // _NBUF, round_, 0)
        for b in range(_NBUF):
            write_chunk(n_chunks - _NBUF + b, b, False)

    return gather_kernel(xidx, table)


_BK = 32
_SLICES = (256, 256, 256, 256)


def _ln_body(g_ref, p_ref, gam_ref, bet_ref, o_ref):
    e = g_ref[...] + p_ref[...]
    m = jnp.mean(e, axis=-1, keepdims=True)
    d = e - m
    v = jnp.mean(d * d, axis=-1, keepdims=True)
    o_ref[...] = d * lax.rsqrt(v + _EPS) * gam_ref[...] + bet_ref[...]


def _tc_layernorm_slice(prev, gathered, pos, gamma, beta, block0, b):
    bs, s, h = gathered.shape

    data_specs = [
        pl.BlockSpec((_BK, s, h), lambda i: (i, 0, 0)),
        pl.BlockSpec((1, s, h), lambda i: (0, 0, 0)),
        pl.BlockSpec((1, 1, h), lambda i: (0, 0, 0)),
        pl.BlockSpec((1, 1, h), lambda i: (0, 0, 0)),
    ]
    common = dict(
        grid=(bs // _BK,),
        out_specs=pl.BlockSpec((_BK, s, h), lambda i: (block0 + i, 0, 0)),
        out_shape=jax.ShapeDtypeStruct((b, s, h), jnp.float32),
    )
    if prev is None:
        return pl.pallas_call(_ln_body, in_specs=data_specs, **common)(
            gathered, pos, gamma, beta)

    def body(_prev_ref, g_ref, p_ref, gam_ref, bet_ref, o_ref):
        _ln_body(g_ref, p_ref, gam_ref, bet_ref, o_ref)

    return pl.pallas_call(
        body,
        in_specs=[pl.BlockSpec((8, 8, h), lambda i: (0, 0, 0))] + data_specs,
        input_output_aliases={0: 0},
        **common,
    )(prev, gathered, pos, gamma, beta)


def kernel(x, word_table, pos_table, ln_gamma, ln_beta):
    b, s = x.shape
    pos = pos_table[:s][None]
    gamma = ln_gamma.reshape(1, 1, _HIDDEN)
    beta = ln_beta.reshape(1, 1, _HIDDEN)

    offs = [0]
    for bs in _SLICES:
        offs.append(offs[-1] + bs)
    gathered = [
        _sc_gather(
            x[offs[i]:offs[i + 1]].reshape(
                _NW, (bs * s) // (_NW * _CHUNK), _CHUNK),
            word_table,
        ).reshape(bs, s, _HIDDEN)
        for i, bs in enumerate(_SLICES)
    ]
    out = None
    for i, bs in enumerate(_SLICES):
        out = _tc_layernorm_slice(
            out, gathered[i], pos, gamma, beta, offs[i] // _BK, b)
    return out


# speedup vs baseline: 1.0914x; 1.0276x over previous
import functools

import jax
import jax.numpy as jnp
from jax import lax
from jax.experimental import pallas as pl
from jax.experimental.pallas import tpu as pltpu
from jax.experimental.pallas import tpu_sc as plsc

_EPS = 1e-12
_HIDDEN = 128


_NC = 2
_NS = 16
_NW = _NC * _NS
_CHUNK = 64


_NBUF = 5


def _sc_gather(xidx, table):
    nw, n_chunks, chunk = xidx.shape
    tok = nw * n_chunks * chunk
    per_w = n_chunks * chunk
    mesh = plsc.VectorSubcoreMesh(core_axis_name="c", subcore_axis_name="s")

    @functools.partial(
        pl.kernel,
        mesh=mesh,
        out_type=jax.ShapeDtypeStruct((tok, _HIDDEN), jnp.float32),
        scratch_types=[
            pltpu.VMEM((n_chunks, chunk), jnp.int32),
            pltpu.VMEM((_NBUF, chunk, _HIDDEN), jnp.float32),
        ]
        + [pltpu.SemaphoreType.DMA] * (2 * _NBUF),
    )
    def gather_kernel(idx_hbm, table_hbm, out_hbm, idx_v, rows_v, *sems):
        gsem, wsem = sems[:_NBUF], sems[_NBUF:]
        wid = lax.axis_index("s") * _NC + lax.axis_index("c")
        base = pl.multiple_of(wid * per_w, chunk)

        pltpu.sync_copy(idx_hbm.at[wid], idx_v)

        def gather_chunk(c, b, start):
            cp = pltpu.make_async_copy(
                table_hbm.at[idx_v.at[c]], rows_v.at[b], gsem[b])
            cp.start() if start else cp.wait()

        def write_chunk(c, b, start):
            off = pl.multiple_of(base + c * chunk, chunk)
            cp = pltpu.make_async_copy(
                rows_v.at[b], out_hbm.at[pl.ds(off, chunk)], wsem[b])
            cp.start() if start else cp.wait()

        for b in range(_NBUF):
            gather_chunk(b, b, True)

        def round_(r, _):
            c0 = r * _NBUF
            for b in range(_NBUF):
                gather_chunk(c0 + b, b, False)
                write_chunk(c0 + b, b, True)
            for b in range(_NBUF):

                @pl.when(r < n_chunks // _NBUF - 1)
                def _():
                    write_chunk(c0 + b, b, False)
                    gather_chunk(c0 + _NBUF + b, b, True)

            return 0

        lax.fori_loop(0, n_chunks // _NBUF, round_, 0)
        for b in range(_NBUF):
            write_chunk(n_chunks - _NBUF + b, b, False)

    return gather_kernel(xidx, table)


_BK = 64
_SLICES = (256, 256, 256, 256)


def _ln_body(g_ref, p_ref, gam_ref, bet_ref, o_ref):
    e = g_ref[...] + p_ref[...]
    m = jnp.mean(e, axis=-1, keepdims=True)
    d = e - m
    v = jnp.mean(d * d, axis=-1, keepdims=True)
    o_ref[...] = d * lax.rsqrt(v + _EPS) * gam_ref[...] + bet_ref[...]


def _tc_layernorm_slice(prev, gathered, pos, gamma, beta, block0, b):
    bs, s, h = gathered.shape

    data_specs = [
        pl.BlockSpec((_BK, s, h), lambda i: (i, 0, 0)),
        pl.BlockSpec((1, s, h), lambda i: (0, 0, 0)),
        pl.BlockSpec((1, 1, h), lambda i: (0, 0, 0)),
        pl.BlockSpec((1, 1, h), lambda i: (0, 0, 0)),
    ]
    common = dict(
        grid=(bs // _BK,),
        out_specs=pl.BlockSpec((_BK, s, h), lambda i: (block0 + i, 0, 0)),
        out_shape=jax.ShapeDtypeStruct((b, s, h), jnp.float32),
    )
    if prev is None:
        return pl.pallas_call(_ln_body, in_specs=data_specs, **common)(
            gathered, pos, gamma, beta)

    def body(_prev_ref, g_ref, p_ref, gam_ref, bet_ref, o_ref):
        _ln_body(g_ref, p_ref, gam_ref, bet_ref, o_ref)

    return pl.pallas_call(
        body,
        in_specs=[pl.BlockSpec((8, 8, h), lambda i: (0, 0, 0))] + data_specs,
        input_output_aliases={0: 0},
        **common,
    )(prev, gathered, pos, gamma, beta)


def kernel(x, word_table, pos_table, ln_gamma, ln_beta):
    b, s = x.shape
    pos = pos_table[:s][None]
    gamma = ln_gamma.reshape(1, 1, _HIDDEN)
    beta = ln_beta.reshape(1, 1, _HIDDEN)

    offs = [0]
    for bs in _SLICES:
        offs.append(offs[-1] + bs)
    gathered = [
        _sc_gather(
            x[offs[i]:offs[i + 1]].reshape(
                _NW, (bs * s) // (_NW * _CHUNK), _CHUNK),
            word_table,
        ).reshape(bs, s, _HIDDEN)
        for i, bs in enumerate(_SLICES)
    ]
    out = None
    for i, bs in enumerate(_SLICES):
        out = _tc_layernorm_slice(
            out, gathered[i], pos, gamma, beta, offs[i] // _BK, b)
    return out
